# baseline (device time: 34363 ns/iter reference)
import jax
import jax.numpy as jnp
from jax import lax
from jax.experimental import pallas as pl
from jax.experimental.pallas import tpu as pltpu

N_DEV = 32
B, SQ, SKV = 2, 128, 128
HQ_LOC, DH = 4, 64
D_MODEL = 512
CHUNK = HQ_LOC * DH
ROWS = B * SQ
SEG = ROWS // N_DEV


def _body(x_ref, wq_hbm, kt_ref, vt_ref, wo_hbm, out_ref, comm_ref,
          wq_v, wo_v, w_sems, rs_send, rs_recv, ag_send, ag_recv):
    me = lax.axis_index("i")
    my_lo = pl.multiple_of(me * SEG, SEG)

    wq_dma = pltpu.make_async_copy(
        wq_hbm.at[:, pl.ds(pl.multiple_of(me * CHUNK, CHUNK), CHUNK)],
        wq_v, w_sems.at[0])
    wo_dma = pltpu.make_async_copy(
        wo_hbm.at[pl.ds(pl.multiple_of(me * CHUNK, CHUNK), CHUNK), :],
        wo_v, w_sems.at[1])
    wq_dma.start()
    wo_dma.start()

    barrier = pltpu.get_barrier_semaphore()
    for k in range(1, N_DEV):
        pl.semaphore_signal(barrier, inc=1, device_id=((me + k) % N_DEV,),
                            device_id_type=pl.DeviceIdType.MESH)
    pl.semaphore_wait(barrier, N_DEV - 1)

    wq_dma.wait()
    wo_dma.wait()
    wq_bf = wq_v[...].astype(jnp.bfloat16)
    wo_bf = wo_v[...].astype(jnp.bfloat16)

    def compute_batch(b):
        xb = x_ref[b].astype(jnp.bfloat16)
        q = lax.dot(xb, wq_bf, preferred_element_type=jnp.float32)
        qb = q.astype(jnp.bfloat16)
        acc = jnp.zeros((SQ, D_MODEL), jnp.float32)
        for h in range(HQ_LOC):
            qh = qb[:, h * DH:(h + 1) * DH]
            kh = kt_ref[b, h]
            s = lax.dot_general(qh, kh, (((1,), (1,)), ((), ())),
                                preferred_element_type=jnp.float32)
            s = s * 0.125
            s = s - jnp.max(s, axis=-1, keepdims=True)
            e = jnp.exp(s)
            w = e / jnp.sum(e, axis=-1, keepdims=True)
            ctx = lax.dot(w.astype(jnp.bfloat16), vt_ref[b, h],
                          preferred_element_type=jnp.float32)
            acc = acc + lax.dot(ctx.astype(jnp.bfloat16),
                                wo_bf[h * DH:(h + 1) * DH, :],
                                preferred_element_type=jnp.float32)
        out_ref[pl.ds(b * SQ, SQ), :] = acc.astype(jnp.bfloat16)

    def rs_send_chunk(c):
        @pl.when(me != c)
        def _():
            rdma = pltpu.make_async_remote_copy(
                src_ref=out_ref.at[pl.ds(c * SEG, SEG), :],
                dst_ref=comm_ref.at[pl.ds(my_lo, SEG), :],
                send_sem=rs_send.at[c],
                recv_sem=rs_recv.at[me],
                device_id=(c,),
                device_id_type=pl.DeviceIdType.MESH,
            )
            rdma.start()

    compute_batch(0)
    for c in range(N_DEV // 2):
        rs_send_chunk(c)
    compute_batch(1)
    for c in range(N_DEV // 2, N_DEV):
        rs_send_chunk(c)

    comm_ref[pl.ds(my_lo, SEG), :] = out_ref[pl.ds(my_lo, SEG), :]

    for s in range(N_DEV):
        @pl.when(me != s)
        def _(s=s):
            d = pltpu.make_async_remote_copy(
                src_ref=comm_ref.at[pl.ds(s * SEG, SEG), :],
                dst_ref=comm_ref.at[pl.ds(s * SEG, SEG), :],
                send_sem=rs_send.at[s],
                recv_sem=rs_recv.at[s],
                device_id=(s,),
                device_id_type=pl.DeviceIdType.MESH,
            )
            d.wait_recv()

    red = jnp.zeros((SEG, D_MODEL), jnp.float32)
    for s in range(N_DEV):
        red = red + comm_ref[pl.ds(s * SEG, SEG), :].astype(jnp.float32)
    out_ref[pl.ds(my_lo, SEG), :] = red.astype(jnp.bfloat16)

    for d_ in range(N_DEV):
        @pl.when(me != d_)
        def _(d_=d_):
            rdma = pltpu.make_async_remote_copy(
                src_ref=out_ref.at[pl.ds(my_lo, SEG), :],
                dst_ref=out_ref.at[pl.ds(my_lo, SEG), :],
                send_sem=ag_send.at[d_],
                recv_sem=ag_recv.at[me],
                device_id=(d_,),
                device_id_type=pl.DeviceIdType.MESH,
            )
            rdma.start()
    for s in range(N_DEV):
        @pl.when(me != s)
        def _(s=s):
            d = pltpu.make_async_remote_copy(
                src_ref=out_ref.at[pl.ds(s * SEG, SEG), :],
                dst_ref=out_ref.at[pl.ds(s * SEG, SEG), :],
                send_sem=ag_send.at[s],
                recv_sem=ag_recv.at[s],
                device_id=(s,),
                device_id_type=pl.DeviceIdType.MESH,
            )
            d.wait_recv()

    for c in range(N_DEV):
        @pl.when(me != c)
        def _(c=c):
            drs = pltpu.make_async_remote_copy(
                src_ref=out_ref.at[pl.ds(c * SEG, SEG), :],
                dst_ref=comm_ref.at[pl.ds(c * SEG, SEG), :],
                send_sem=rs_send.at[c],
                recv_sem=rs_recv.at[c],
                device_id=(c,),
                device_id_type=pl.DeviceIdType.MESH,
            )
            drs.wait_send()
            dag = pltpu.make_async_remote_copy(
                src_ref=out_ref.at[pl.ds(my_lo, SEG), :],
                dst_ref=out_ref.at[pl.ds(my_lo, SEG), :],
                send_sem=ag_send.at[c],
                recv_sem=ag_recv.at[c],
                device_id=(c,),
                device_id_type=pl.DeviceIdType.MESH,
            )
            dag.wait_send()


def kernel(x, Wq, K_ext, V_ext, Wo):
    kt = jnp.transpose(K_ext, (0, 2, 1, 3)).astype(jnp.bfloat16)
    vt = jnp.transpose(V_ext, (0, 2, 1, 3)).astype(jnp.bfloat16)

    out2d = pl.pallas_call(
        _body,
        out_shape=jax.ShapeDtypeStruct((ROWS, D_MODEL), jnp.bfloat16),
        in_specs=[
            pl.BlockSpec(memory_space=pltpu.VMEM),
            pl.BlockSpec(memory_space=pltpu.MemorySpace.HBM),
            pl.BlockSpec(memory_space=pltpu.VMEM),
            pl.BlockSpec(memory_space=pltpu.VMEM),
            pl.BlockSpec(memory_space=pltpu.MemorySpace.HBM),
        ],
        out_specs=pl.BlockSpec(memory_space=pltpu.VMEM),
        scratch_shapes=[
            pltpu.VMEM((ROWS, D_MODEL), jnp.bfloat16),
            pltpu.VMEM((D_MODEL, CHUNK), jnp.float32),
            pltpu.VMEM((CHUNK, D_MODEL), jnp.float32),
            pltpu.SemaphoreType.DMA((2,)),
            pltpu.SemaphoreType.DMA((N_DEV,)),
            pltpu.SemaphoreType.DMA((N_DEV,)),
            pltpu.SemaphoreType.DMA((N_DEV,)),
            pltpu.SemaphoreType.DMA((N_DEV,)),
        ],
        compiler_params=pltpu.CompilerParams(collective_id=0),
    )(x, Wq, kt, vt, Wo)
    return out2d.reshape(B, SQ, D_MODEL)
